# baseline (speedup 1.0000x reference)
import functools

import jax
import jax.numpy as jnp
from jax import lax
from jax.experimental import pallas as pl
from jax.experimental.pallas import tpu as pltpu
from jax.experimental.pallas import tpu_sc as plsc

B = 16384
F = 26
V = 100000
D = 32
C = 16
H0, H1, H2 = 512, 256, 128
FD = F * D

NC, NS = 2, 16
NW = NC * NS
TR = F * V
PR = TR // 4

K1_CH = 128
K1_RPW = 81280
K1_LAST = TR - (NW - 1) * K1_RPW

BPW = B // NW
IPW = BPW * F
GRP = 32
NG = BPW // GRP
NI = NG * F
RING = 8

_mesh = plsc.VectorSubcoreMesh(core_axis_name="c", subcore_axis_name="s")


@functools.partial(
    pl.kernel,
    out_type=jax.ShapeDtypeStruct((PR, 128), jnp.float32),
    mesh=_mesh,
    scratch_types=(
        [pltpu.VMEM((K1_CH, 32), jnp.float32)] * 4
        + [pltpu.VMEM((K1_CH // 4, 128), jnp.float32)] * 4
        + [pltpu.SemaphoreType.DMA] * 8
    ),
)
def _k1(tab, out, *s):
    b32s, bps, srs, sws = s[0:4], s[4:8], s[8:12], s[12:16]
    wid = lax.axis_index("s") * NC + lax.axis_index("c")
    wb = wid * K1_RPW
    rows_w = jnp.where(wid < NW - 1, K1_RPW, K1_LAST)
    end = wb + rows_w
    n_c = (rows_w + K1_CH - 1) // K1_CH
    n_blk = (n_c + 3) // 4
    n_it = n_blk * 4

    def base(c):
        return pl.multiple_of(jnp.minimum(wb + c * K1_CH, end - K1_CH), 32)

    for st in range(4):
        pltpu.async_copy(tab.at[pl.ds(base(st), K1_CH), :], b32s[st], srs[st])

    def blk_body(blk, carry):
        for st in range(4):
            b32, bp, sr, sw = b32s[st], bps[st], srs[st], sws[st]
            c = blk * 4 + st
            pltpu.make_async_copy(
                tab.at[pl.ds(base(c), K1_CH), :], b32, sr).wait()

            @pl.when(c >= 4)
            def _():
                pltpu.make_async_copy(
                    bp, out.at[pl.ds(0, K1_CH // 4), :], sw).wait()

            def vbody(v, cy):
                for j in range(8):
                    bp[v, pl.ds(j * 16, 16)] = (
                        b32[4 * v + j // 2, pl.ds((j % 2) * 16, 16)])
                return cy

            lax.fori_loop(0, K1_CH // 4, vbody, 0)
            pltpu.async_copy(bp, out.at[pl.ds(pl.multiple_of(base(c) // 4, 8), K1_CH // 4), :], sw)

            @pl.when(c + 4 < n_it)
            def _():
                pltpu.async_copy(
                    tab.at[pl.ds(base(c + 4), K1_CH), :], b32, sr)
        return carry

    lax.fori_loop(0, n_blk, blk_body, 0)
    for st in range(4):
        pltpu.make_async_copy(
            bps[st], out.at[pl.ds(0, K1_CH // 4), :], sws[st]).wait()


@functools.partial(
    pl.kernel,
    out_type=jax.ShapeDtypeStruct((B, FD), jnp.float32),
    mesh=_mesh,
    scratch_types=(
        [pltpu.VMEM((IPW,), jnp.int32)] * 2
        + [pltpu.VMEM((GRP, 128), jnp.float32)] * RING
        + [pltpu.VMEM((GRP, FD), jnp.float32)]
        + [pltpu.SemaphoreType.DMA] * RING
    ),
    compiler_params=pltpu.CompilerParams(needs_layout_passes=False),
)
def _k2(tab128, quad, qb, out, *s):
    quad_v, qb_v = s[0], s[1]
    gbufs, asm, sems = s[2:2 + RING], s[2 + RING], s[3 + RING:3 + 2 * RING]
    wid = lax.axis_index("s") * NC + lax.axis_index("c")
    wb = pl.multiple_of(wid * BPW, 32)
    ib = pl.multiple_of(wid * IPW, 8)
    pltpu.sync_copy(quad.at[pl.ds(ib, IPW)], quad_v)
    pltpu.sync_copy(qb.at[pl.ds(ib, IPW)], qb_v)

    def issue(f, g, buf, sem):
        pltpu.async_copy(
            tab128.at[quad_v.at[pl.ds(pl.multiple_of(f * BPW + g * GRP, 8), GRP)]], buf, sem)

    for st in range(RING):
        issue(st % F, st // F, gbufs[st], sems[st])

    iota = lax.iota(jnp.int32, 16)

    def blk_body(blk, carry):
        for st in range(RING):
            i = blk * RING + st
            f = i % F
            g = i // F
            buf, sem = gbufs[st], sems[st]
            pltpu.make_async_copy(
                tab128.at[quad_v.at[pl.ds(pl.multiple_of(f * BPW + g * GRP, 8), GRP)]],
                buf, sem).wait()
            for rg in range(2):
                rows = rg * 16 + iota
                qbv = plsc.load_gather(qb_v, [f * BPW + g * GRP + rows])
                for j in range(32):
                    v = plsc.load_gather(buf, [rows, qbv + j])
                    plsc.store_scatter(
                        asm, [rows, jnp.full((16,), f * 32 + j, jnp.int32)], v)

            @pl.when(i + RING < NI)
            def _():
                i2 = i + RING
                issue(i2 % F, i2 // F, buf, sem)

            @pl.when(f == F - 1)
            def _():
                pltpu.sync_copy(asm, out.at[pl.ds(pl.multiple_of(wb + g * GRP, 8), GRP), :])
        return carry

    lax.fori_loop(0, NI // RING, blk_body, 0)


BM = 512


def _mlp_body(emb_ref, cont_ref, w0a_ref, w0b_ref, b0_ref, w1_ref, b1_ref,
              w2_ref, b2_ref, out_ref):
    h0 = jnp.dot(emb_ref[...], w0a_ref[...], preferred_element_type=jnp.float32)
    h0 += jnp.dot(cont_ref[...], w0b_ref[...], preferred_element_type=jnp.float32)
    h0 = jnp.maximum(h0 + b0_ref[...], 0.0)
    h1 = jnp.maximum(
        jnp.dot(h0, w1_ref[...], preferred_element_type=jnp.float32) + b1_ref[...], 0.0)
    out_ref[...] = jnp.maximum(
        jnp.dot(h1, w2_ref[...], preferred_element_type=jnp.float32) + b2_ref[...], 0.0)


_mlp = pl.pallas_call(
    _mlp_body,
    grid=(B // BM,),
    in_specs=[
        pl.BlockSpec((BM, FD), lambda i: (i, 0)),
        pl.BlockSpec((BM, C), lambda i: (i, 0)),
        pl.BlockSpec((FD, H0), lambda i: (0, 0)),
        pl.BlockSpec((C, H0), lambda i: (0, 0)),
        pl.BlockSpec((1, H0), lambda i: (0, 0)),
        pl.BlockSpec((H0, H1), lambda i: (0, 0)),
        pl.BlockSpec((1, H1), lambda i: (0, 0)),
        pl.BlockSpec((H1, H2), lambda i: (0, 0)),
        pl.BlockSpec((1, H2), lambda i: (0, 0)),
    ],
    out_specs=pl.BlockSpec((BM, H2), lambda i: (i, 0)),
    out_shape=jax.ShapeDtypeStruct((B, H2), jnp.float32),
)


def kernel(continuous, categorical_indices, tables, W0, b0, W1, b1, W2, b2):
    offsets = (jnp.arange(F, dtype=jnp.int32) * V)[None, :]
    flat = categorical_indices + offsets
    t3 = flat.reshape(NW, BPW, F).transpose(0, 2, 1)
    quad = (t3 >> 2).reshape(-1)
    qb = ((t3 & 3) << 5).reshape(-1)
    tab128 = _k1(tables)
    emb = _k2(tab128, quad, qb)
    return _mlp(emb, continuous.astype(jnp.float32),
                W0[:FD], W0[FD:],
                b0.reshape(1, H0), W1, b1.reshape(1, H1),
                W2, b2.reshape(1, H2))
